# baseline (speedup 1.0000x reference)
import functools

import jax
import jax.numpy as jnp
from jax import lax
from jax.experimental import pallas as pl
from jax.experimental.pallas import tpu as pltpu
from jax.experimental.pallas import tpu_sc as plsc

VOCAB = 100000
EMBED = 64
SEQ = 50
ROWS = 4096
LANES = 16

NC = 2
NS = 16
NW = NC * NS

BBLK = ROWS // NW
NBUF = 5
TPAD = 133

_mesh = plsc.VectorSubcoreMesh(core_axis_name="c", subcore_axis_name="s")


@functools.partial(
    pl.kernel,
    mesh=_mesh,
    out_type=jax.ShapeDtypeStruct((SEQ, EMBED // 8, NW, 8, BBLK), jnp.float32),
    compiler_params=pltpu.CompilerParams(use_tc_tiling_on_sc=False, needs_layout_passes=False),
    scratch_types=[
        pltpu.VMEM((SEQ, BBLK), jnp.int32),
        pltpu.VMEM((NBUF, BBLK, EMBED), jnp.float32),
        pltpu.VMEM((NBUF, EMBED // 8, 8, TPAD), jnp.float32),
        *([pltpu.SemaphoreType.DMA] * NBUF),
        *([pltpu.SemaphoreType.DMA] * NBUF),
    ],
)
def _emb_lookup(idx_hbm, table_hbm, out_hbm, idx_v, gbuf, tbuf, *sems):
    gsem = sems[:NBUF]
    ssem = sems[NBUF:]
    wid = lax.axis_index("s") * NC + lax.axis_index("c")
    pltpu.sync_copy(idx_hbm.at[:, pl.ds(wid * BBLK, BBLK)], idx_v)

    lane = lax.iota(jnp.int32, LANES)

    def gather_start(s, b):
        pltpu.async_copy(table_hbm.at[idx_v.at[s]], gbuf.at[b], gsem[b])

    def gather_wait(s, b):
        pltpu.make_async_copy(
            table_hbm.at[idx_v.at[s]], gbuf.at[b], gsem[b]).wait()

    def scatter_start(s, b):
        pltpu.async_copy(tbuf.at[b, :, :, pl.ds(0, BBLK)],
                         out_hbm.at[s, :, wid], ssem[b])

    def scatter_wait(s, b):
        pltpu.make_async_copy(tbuf.at[b, :, :, pl.ds(0, BBLK)],
                              out_hbm.at[s, :, wid], ssem[b]).wait()

    NE0 = EMBED // LANES
    ti_static = [(lane + e0 * LANES) // 8 for e0 in range(NE0)]
    r_static = [(lane + e0 * LANES) % 8 for e0 in range(NE0)]
    bsl_static = [jnp.full((LANES,), b, dtype=jnp.int32) for b in range(NBUF)]

    def transpose_block(b):
        bsl = bsl_static[b]

        @plsc.parallel_loop(0, BBLK, unroll=8)
        def brow(c):
            cols = jnp.full((LANES,), c, dtype=jnp.int32)
            for e0 in range(NE0):
                v = gbuf[b, c, pl.ds(e0 * LANES, LANES)]
                plsc.store_scatter(
                    tbuf, [bsl, ti_static[e0], r_static[e0], cols], v)

    for b in range(NBUF):
        gather_start(b, b)
    for s in range(NBUF):
        b = s % NBUF
        gather_wait(s, b)
        transpose_block(b)
        scatter_start(s, b)
        gather_start(s + NBUF, b)

    def rounds(i, carry):
        s0 = i * NBUF
        for b in range(NBUF):
            s = s0 + b
            gather_wait(s, b)
            scatter_wait(s - NBUF, b)
            transpose_block(b)
            scatter_start(s, b)
            gather_start(s + NBUF, b)
        return carry

    lax.fori_loop(1, SEQ // NBUF - 1, rounds, 0)

    for s in range(SEQ - NBUF, SEQ):
        b = s % NBUF
        gather_wait(s, b)
        scatter_wait(s - NBUF, b)
        transpose_block(b)
        scatter_start(s, b)
    for s in range(SEQ - NBUF, SEQ):
        scatter_wait(s, s % NBUF)


def kernel(x, table):
    buf = _emb_lookup(x.T.astype(jnp.int32), table)
    return buf.transpose(2, 4, 0, 1, 3).reshape(ROWS, SEQ, EMBED)


# speedup vs baseline: 1.0168x; 1.0168x over previous
import functools

import jax
import jax.numpy as jnp
from jax import lax
from jax.experimental import pallas as pl
from jax.experimental.pallas import tpu as pltpu
from jax.experimental.pallas import tpu_sc as plsc

VOCAB = 100000
EMBED = 64
SEQ = 50
ROWS = 4096
LANES = 16

NC = 2
NS = 16
NW = NC * NS

BBLK = ROWS // NW
NBUF = 5
TPAD = 133

_mesh = plsc.VectorSubcoreMesh(core_axis_name="c", subcore_axis_name="s")


@functools.partial(
    pl.kernel,
    mesh=_mesh,
    out_type=jax.ShapeDtypeStruct((SEQ, EMBED // 8, NW, 8, BBLK), jnp.float32),
    compiler_params=pltpu.CompilerParams(use_tc_tiling_on_sc=False, needs_layout_passes=False),
    scratch_types=[
        pltpu.VMEM((SEQ, BBLK), jnp.int32),
        pltpu.VMEM((NBUF, BBLK, EMBED), jnp.float32),
        pltpu.VMEM((NBUF, EMBED // 8, 8, TPAD), jnp.float32),
        *([pltpu.SemaphoreType.DMA] * NBUF),
        *([pltpu.SemaphoreType.DMA] * NBUF),
    ],
)
def _emb_lookup(idx_hbm, table_hbm, out_hbm, idx_v, gbuf, tbuf, *sems):
    gsem = sems[:NBUF]
    ssem = sems[NBUF:]
    wid = lax.axis_index("s") * NC + lax.axis_index("c")
    pltpu.sync_copy(idx_hbm.at[:, pl.ds(wid * BBLK, BBLK)], idx_v)

    lane = lax.iota(jnp.int32, LANES)

    def gather_start(s, b):
        pltpu.async_copy(table_hbm.at[idx_v.at[s]], gbuf.at[b], gsem[b])

    def gather_wait(s, b):
        pltpu.make_async_copy(
            table_hbm.at[idx_v.at[s]], gbuf.at[b], gsem[b]).wait()

    def scatter_start(s, b):
        pltpu.async_copy(tbuf.at[b, :, :, pl.ds(0, BBLK)],
                         out_hbm.at[s, :, wid], ssem[b])

    def scatter_wait(s, b):
        pltpu.make_async_copy(tbuf.at[b, :, :, pl.ds(0, BBLK)],
                              out_hbm.at[s, :, wid], ssem[b]).wait()

    NE0 = EMBED // LANES
    ti_static = [(lane + e0 * LANES) // 8 for e0 in range(NE0)]
    r_static = [(lane + e0 * LANES) % 8 for e0 in range(NE0)]
    bsl_static = [jnp.full((LANES,), b, dtype=jnp.int32) for b in range(NBUF)]

    def transpose_block(b):
        bsl = bsl_static[b]

        @plsc.parallel_loop(0, BBLK, unroll=4)
        def brow(c):
            cols = jnp.full((LANES,), c, dtype=jnp.int32)
            for e0 in range(NE0):
                v = gbuf[b, c, pl.ds(e0 * LANES, LANES)]
                plsc.store_scatter(
                    tbuf, [bsl, ti_static[e0], r_static[e0], cols], v)

    for b in range(NBUF):
        gather_start(b, b)
    for s in range(NBUF):
        b = s % NBUF
        gather_wait(s, b)
        transpose_block(b)
        scatter_start(s, b)
        gather_start(s + NBUF, b)

    def rounds(i, carry):
        s0 = i * NBUF
        for b in range(NBUF):
            s = s0 + b
            gather_wait(s, b)
            scatter_wait(s - NBUF, b)
            transpose_block(b)
            scatter_start(s, b)
            gather_start(s + NBUF, b)
        return carry

    lax.fori_loop(1, SEQ // NBUF - 1, rounds, 0)

    for s in range(SEQ - NBUF, SEQ):
        b = s % NBUF
        gather_wait(s, b)
        scatter_wait(s - NBUF, b)
        transpose_block(b)
        scatter_start(s, b)
    for s in range(SEQ - NBUF, SEQ):
        scatter_wait(s, s % NBUF)


def kernel(x, table):
    buf = _emb_lookup(x.T.astype(jnp.int32), table)
    return buf.transpose(2, 4, 0, 1, 3).reshape(ROWS, SEQ, EMBED)
